# baseline (speedup 1.0000x reference)
import functools

import jax
import jax.numpy as jnp
from jax import lax
from jax.experimental import pallas as pl
from jax.experimental.pallas import tpu as pltpu
from jax.experimental.pallas import tpu_sc as plsc

N_TRACKS_C = 1000000
D_MODEL_C = 64
BATCH_C = 16384

NUM_CORES = 2
NUM_SUBCORES = 16
LANES = 16
NUM_WORKERS = NUM_CORES * NUM_SUBCORES
B_PER_W = BATCH_C // NUM_WORKERS
GROUPS = B_PER_W // LANES


def _body(users_hbm, tracks_hbm, first_hbm, ub0_hbm, ub1_hbm, table_hbm,
          out_hbm,
          uidx_v, tidx_v, fidx_v, t_rows, f_rows, ub0_v, ub1_v, out_v,
          sem_t, sem_f, sem_u0, sem_u1):
    wid = lax.axis_index("s") * NUM_CORES + lax.axis_index("c")
    base = wid * B_PER_W

    pltpu.sync_copy(tracks_hbm.at[pl.ds(base, B_PER_W)], tidx_v)
    pltpu.sync_copy(first_hbm.at[pl.ds(base, B_PER_W)], fidx_v)
    pltpu.sync_copy(users_hbm.at[pl.ds(base, B_PER_W)], uidx_v)

    ct = pltpu.async_copy(table_hbm.at[tidx_v], t_rows, sem_t)
    cf = pltpu.async_copy(table_hbm.at[fidx_v], f_rows, sem_f)
    c0 = pltpu.async_copy(ub0_hbm.at[uidx_v], ub0_v, sem_u0)
    c1 = pltpu.async_copy(ub1_hbm.at[uidx_v], ub1_v, sem_u1)
    ct.wait()
    cf.wait()
    c0.wait()
    c1.wait()

    iota = lax.iota(jnp.int32, LANES)

    def group_body(g, carry):
        rows = g * LANES + iota

        def d_body(d, acc):
            dcol = jnp.full((LANES,), d, jnp.int32)
            fv = plsc.load_gather(f_rows, [rows, dcol])
            tv = plsc.load_gather(t_rows, [rows, dcol])
            return acc + fv * tv

        acc = lax.fori_loop(0, D_MODEL_C, d_body,
                            jnp.zeros((LANES,), jnp.float32))
        b0 = ub0_v[pl.ds(g * LANES, LANES)]
        b1 = ub1_v[pl.ds(g * LANES, LANES)]
        x = (acc - b0) * b1
        y = 1.0 / (1.0 + jnp.exp(-x))
        out_v[pl.ds(g * LANES, LANES)] = y
        return carry

    lax.fori_loop(0, GROUPS, group_body, 0)
    pltpu.sync_copy(out_v, out_hbm.at[pl.ds(base, B_PER_W)])


@jax.jit
def _run(users, tracks, first_tracks, ub0, ub1, tracks_table):
    mesh = plsc.VectorSubcoreMesh(core_axis_name="c", subcore_axis_name="s")
    f = functools.partial(
        pl.kernel,
        out_type=jax.ShapeDtypeStruct((BATCH_C,), jnp.float32),
        mesh=mesh,
        compiler_params=pltpu.CompilerParams(
            needs_layout_passes=False, use_tc_tiling_on_sc=False),
        scratch_types=[
            pltpu.VMEM((B_PER_W,), jnp.int32),
            pltpu.VMEM((B_PER_W,), jnp.int32),
            pltpu.VMEM((B_PER_W,), jnp.int32),
            pltpu.VMEM((B_PER_W, D_MODEL_C), jnp.float32),
            pltpu.VMEM((B_PER_W, D_MODEL_C), jnp.float32),
            pltpu.VMEM((B_PER_W,), jnp.float32),
            pltpu.VMEM((B_PER_W,), jnp.float32),
            pltpu.VMEM((B_PER_W,), jnp.float32),
            pltpu.SemaphoreType.DMA,
            pltpu.SemaphoreType.DMA,
            pltpu.SemaphoreType.DMA,
            pltpu.SemaphoreType.DMA,
        ],
    )(_body)
    return f(users, tracks, first_tracks, ub0, ub1, tracks_table)


def kernel(users, tracks, first_tracks, user_bias, tracks_table):
    users = users.astype(jnp.int32)
    tracks = tracks.astype(jnp.int32)
    first_tracks = first_tracks.astype(jnp.int32)
    ub0 = user_bias[0]
    ub1 = user_bias[1]
    return _run(users, tracks, first_tracks, ub0, ub1, tracks_table)
